# baseline (speedup 1.0000x reference)
import functools

import jax
import jax.numpy as jnp
from jax import lax
from jax.experimental import pallas as pl
from jax.experimental.pallas import tpu as pltpu
from jax.experimental.pallas import tpu_sc as plsc

S = 384
CZ = 256
VBINS = 2 * (S - 1) + 1
VPAD = 768


def _emb_body(w_ref, b_ref, out_ref):
    out_ref[...] = w_ref[...].T + b_ref[...]


def _build_emb(w_pad, b2):
    return pl.pallas_call(
        _emb_body,
        out_shape=jax.ShapeDtypeStruct((VPAD, CZ), jnp.float32),
    )(w_pad, b2)


def _make_sc_gather():
    info = plsc.get_sparse_core_info()
    nc, ns = info.num_cores, info.num_subcores
    nw = nc * ns
    b_total = S * S
    b_per_w = b_total // nw
    chunk = 128
    n_chunks = b_per_w // chunk
    mesh = plsc.VectorSubcoreMesh(core_axis_name="c", subcore_axis_name="s")

    @functools.partial(
        pl.kernel,
        mesh=mesh,
        out_type=jax.ShapeDtypeStruct((b_total, CZ), jnp.float32),
        scratch_types=[
            pltpu.VMEM((n_chunks, chunk), jnp.int32),
            pltpu.VMEM((chunk, CZ), jnp.float32),
            pltpu.SemaphoreType.DMA,
        ],
    )
    def sc_gather(emb_hbm, idx_hbm, out_hbm, idx_v, buf, sem):
        wid = lax.axis_index("s") * nc + lax.axis_index("c")
        base = wid * b_per_w
        pltpu.sync_copy(idx_hbm.at[wid], idx_v)

        def body(g, carry):
            pltpu.async_copy(emb_hbm.at[idx_v.at[g]], buf, sem).wait()
            pltpu.sync_copy(buf, out_hbm.at[pl.ds(base + g * chunk, chunk)])
            return carry

        lax.fori_loop(0, n_chunks, body, 0)

    return sc_gather, nw, n_chunks, chunk


_SC_GATHER, _NW, _NCHUNKS, _CHUNK = None, None, None, None


def _get_sc_gather():
    global _SC_GATHER, _NW, _NCHUNKS, _CHUNK
    if _SC_GATHER is None:
        _SC_GATHER, _NW, _NCHUNKS, _CHUNK = _make_sc_gather()
    return _SC_GATHER


def kernel(seq_len, ResInd, Wp_w, Wp_b):
    sc_gather = _get_sc_gather()
    s = ResInd.shape[0]
    start = seq_len - s
    ri = lax.dynamic_slice(ResInd, (start, start), (s, s))
    idx = ri.reshape(_NW, _NCHUNKS, _CHUNK).astype(jnp.int32)
    w_pad = jnp.pad(Wp_w, ((0, 0), (0, VPAD - VBINS)))
    emb = _build_emb(w_pad, Wp_b.reshape(1, CZ))
    out = sc_gather(emb, idx)
    return out.reshape(s, s, CZ)
